# baseline (speedup 1.0000x reference)
import functools

import jax
import jax.numpy as jnp
import numpy as np
from jax import lax
from jax.experimental import pallas as pl
from jax.experimental.pallas import tpu as pltpu
from jax.experimental.pallas import tpu_sc as plsc

N = 10000
E = 320000
H = 128
OUT = 128
EF = 16
NSIG = 15
CB = 128
BN = 2000
BE = 2000


def _tables_body(f_ref, ws_ref, wd_ref, ta_ref, tb_ref):
    f = f_ref[...]
    ta_ref[...] = jnp.dot(f, ws_ref[...], preferred_element_type=jnp.float32)
    tb_ref[...] = jnp.dot(f, wd_ref[...], preferred_element_type=jnp.float32)


def _build_tables(feats, w1s, w1d):
    return pl.pallas_call(
        _tables_body,
        grid=(N // BN,),
        in_specs=[
            pl.BlockSpec((BN, H), lambda i: (i, 0)),
            pl.BlockSpec((H, H), lambda i: (0, 0)),
            pl.BlockSpec((H, H), lambda i: (0, 0)),
        ],
        out_specs=[
            pl.BlockSpec((BN, H), lambda i: (i, 0)),
            pl.BlockSpec((BN, H), lambda i: (i, 0)),
        ],
        out_shape=[
            jax.ShapeDtypeStruct((N, H), jnp.float32),
            jax.ShapeDtypeStruct((N, H), jnp.float32),
        ],
    )(feats, w1s, w1d)


@functools.lru_cache(maxsize=None)
def _make_gather():
    info = plsc.get_sparse_core_info()
    nc, ns = info.num_cores, info.num_subcores
    nw = nc * ns
    nchg = E // CB

    def _gather_body(ta, tb, c4, src, dst, g_out, d2_out,
                     idx_s, idx_d, rows_a, rows_b, d2_v, c4_v, gsem, ssem):
        wid = lax.axis_index("s") * nc + lax.axis_index("c")
        nloc = (nchg - 1 - wid) // nw + 1
        pltpu.sync_copy(c4, c4_v)

        def load_and_fire(k):
            b2 = k % 2
            a3 = k % 3
            e0 = (wid + k * nw) * CB
            pltpu.sync_copy(src.at[pl.ds(e0, CB)], idx_s.at[b2])
            pltpu.sync_copy(dst.at[pl.ds(e0, CB)], idx_d.at[b2])
            pltpu.async_copy(ta.at[idx_s.at[b2]], rows_a.at[a3], gsem)
            pltpu.async_copy(tb.at[idx_d.at[b2]], rows_b.at[b2], gsem)

        load_and_fire(0)

        def chunk(k, carry):
            b2 = k % 2
            a3 = k % 3
            e0 = (wid + k * nw) * CB

            @pl.when(k + 1 < nloc)
            def _():
                @pl.when(k >= 2)
                def _():
                    pltpu.make_async_copy(
                        g_out.at[pl.ds(0, CB)], rows_a.at[0], ssem).wait()
                    pltpu.make_async_copy(
                        d2_out.at[pl.ds(0, CB)], d2_v.at[0], ssem).wait()
                load_and_fire(k + 1)

            for gblk in range(CB // 16):
                sl = pl.ds(gblk * 16, 16)
                is4 = idx_s[b2, sl] << 2
                id4 = idx_d[b2, sl] << 2
                dx = (plsc.load_gather(c4_v, [is4])
                      - plsc.load_gather(c4_v, [id4]))
                dy = (plsc.load_gather(c4_v, [is4 + 1])
                      - plsc.load_gather(c4_v, [id4 + 1]))
                dz = (plsc.load_gather(c4_v, [is4 + 2])
                      - plsc.load_gather(c4_v, [id4 + 2]))
                d2_v[a3, sl] = dx * dx + dy * dy + dz * dz

            pltpu.make_async_copy(ta.at[pl.ds(0, CB)], rows_a.at[0], gsem).wait()
            pltpu.make_async_copy(ta.at[pl.ds(0, CB)], rows_b.at[0], gsem).wait()

            def row(i, carry2):
                for j in range(H // 16):
                    sl = pl.ds(j * 16, 16)
                    rows_a[a3, i, sl] = rows_a[a3, i, sl] + rows_b[b2, i, sl]
                return carry2

            lax.fori_loop(0, CB, row, 0)
            pltpu.async_copy(rows_a.at[a3], g_out.at[pl.ds(e0, CB)], ssem)
            pltpu.async_copy(d2_v.at[a3], d2_out.at[pl.ds(e0, CB)], ssem)
            return carry

        lax.fori_loop(0, nloc, chunk, 0)

        for _ in range(3):
            pltpu.make_async_copy(
                g_out.at[pl.ds(0, CB)], rows_a.at[0], ssem).wait()
            pltpu.make_async_copy(
                d2_out.at[pl.ds(0, CB)], d2_v.at[0], ssem).wait()

    return functools.partial(
        pl.kernel,
        out_type=[
            jax.ShapeDtypeStruct((E, H), jnp.float32),
            jax.ShapeDtypeStruct((E,), jnp.float32),
        ],
        mesh=plsc.VectorSubcoreMesh(core_axis_name="c", subcore_axis_name="s"),
        compiler_params=pltpu.CompilerParams(needs_layout_passes=False),
        scratch_types=[
            pltpu.VMEM((2, CB), jnp.int32),
            pltpu.VMEM((2, CB), jnp.int32),
            pltpu.VMEM((3, CB, H), jnp.float32),
            pltpu.VMEM((2, CB, H), jnp.float32),
            pltpu.VMEM((3, CB), jnp.float32),
            pltpu.VMEM((4 * N,), jnp.float32),
            pltpu.SemaphoreType.DMA,
            pltpu.SemaphoreType.DMA,
        ],
    )(_gather_body)


def _mlp_body(g_ref, d2_ref, ef_ref, w1e_ref, w1m_ref, isg_ref, b1_ref,
              lng_ref, lnb_ref, w2_ref, b2_ref, o_ref):
    d2 = d2_ref[...]
    mag = jnp.exp(-d2 * isg_ref[...])
    u = (g_ref[...]
         + jnp.dot(ef_ref[...], w1e_ref[...], preferred_element_type=jnp.float32)
         + jnp.dot(mag, w1m_ref[...], preferred_element_type=jnp.float32)
         + b1_ref[...])
    h = jnp.where(u > 0, u, 0.01 * u)
    mu = jnp.mean(h, axis=-1, keepdims=True)
    hc = h - mu
    var = jnp.mean(hc * hc, axis=-1, keepdims=True)
    hn = hc * lax.rsqrt(var + 1e-5) * lng_ref[...] + lnb_ref[...]
    o_ref[...] = jnp.dot(hn, w2_ref[...], preferred_element_type=jnp.float32) + b2_ref[...]


def _mlp(g, d2, ef, w1e, w1m, isg, b1, lng, lnb, w2, b2):
    full = lambda i: (0, 0)
    return pl.pallas_call(
        _mlp_body,
        grid=(E // BE,),
        in_specs=[
            pl.BlockSpec((BE, H), lambda i: (i, 0)),
            pl.BlockSpec((BE, 1), lambda i: (i, 0)),
            pl.BlockSpec((BE, EF), lambda i: (i, 0)),
            pl.BlockSpec((EF, OUT), full),
            pl.BlockSpec((16, OUT), full),
            pl.BlockSpec((1, 16), full),
            pl.BlockSpec((1, OUT), full),
            pl.BlockSpec((1, OUT), full),
            pl.BlockSpec((1, OUT), full),
            pl.BlockSpec((OUT, OUT), full),
            pl.BlockSpec((1, OUT), full),
        ],
        out_specs=pl.BlockSpec((BE, OUT), lambda i: (i, 0)),
        out_shape=jax.ShapeDtypeStruct((E, OUT), jnp.float32),
    )(g, d2, ef, w1e, w1m, isg, b1, lng, lnb, w2, b2)


def kernel(coors_ligand, h_feats_ligand, original_ligand_node_features,
           original_edge_feats_ligand, orig_coors_ligand, coors_receptor,
           h_feats_receptor, original_receptor_node_features,
           original_edge_feats_receptor, orig_coors_receptor,
           edge_index_ligand, edge_index_receptor, W1, b1, ln_g, ln_b, W2, b2):
    w1s = W1[:H]
    w1d = W1[H:2 * H]
    w1e = W1[2 * H:2 * H + EF]
    w1m = jnp.concatenate([W1[2 * H + EF:], jnp.zeros((1, OUT), jnp.float32)], axis=0)
    isg = jnp.asarray(
        np.concatenate([1.0 / (1.5 ** np.arange(NSIG)), [0.0]]), jnp.float32
    ).reshape(1, 16)
    b1r = b1.reshape(1, OUT)
    b2r = b2.reshape(1, OUT)
    lngr = ln_g.reshape(1, OUT)
    lnbr = ln_b.reshape(1, OUT)

    c4_l = jnp.pad(coors_ligand, ((0, 0), (0, 1))).reshape(4 * N)
    c4_r = jnp.pad(coors_receptor, ((0, 0), (0, 1))).reshape(4 * N)

    ta_l, tb_l = _build_tables(h_feats_ligand, w1s, w1d)
    ta_r, tb_r = _build_tables(h_feats_receptor, w1s, w1d)

    gather = _make_gather()
    g_l, d2_l = gather(ta_l, tb_l, c4_l, edge_index_ligand[0], edge_index_ligand[1])
    g_r, d2_r = gather(ta_r, tb_r, c4_r, edge_index_receptor[0], edge_index_receptor[1])

    msg_ll = _mlp(g_l, d2_l.reshape(E, 1), original_edge_feats_ligand,
                  w1e, w1m, isg, b1r, lngr, lnbr, W2, b2r)
    msg_rr = _mlp(g_r, d2_r.reshape(E, 1), original_edge_feats_receptor,
                  w1e, w1m, isg, b1r, lngr, lnbr, W2, b2r)
    return (msg_ll, msg_rr)


# speedup vs baseline: 1.3572x; 1.3572x over previous
import functools

import jax
import jax.numpy as jnp
import numpy as np
from jax import lax
from jax.experimental import pallas as pl
from jax.experimental.pallas import tpu as pltpu
from jax.experimental.pallas import tpu_sc as plsc

N = 10000
E = 320000
H = 128
OUT = 128
EF = 16
NSIG = 15
CB = 80
BN = 2000
BE = 2000


def _tables_body(f_ref, ws_ref, wd_ref, ta_ref, tb_ref):
    f = f_ref[...]
    ta_ref[...] = jnp.dot(f, ws_ref[...], preferred_element_type=jnp.float32)
    tb_ref[...] = jnp.dot(f, wd_ref[...], preferred_element_type=jnp.float32)


def _build_tables(feats, w1s, w1d):
    return pl.pallas_call(
        _tables_body,
        grid=(N // BN,),
        in_specs=[
            pl.BlockSpec((BN, H), lambda i: (i, 0)),
            pl.BlockSpec((H, H), lambda i: (0, 0)),
            pl.BlockSpec((H, H), lambda i: (0, 0)),
        ],
        out_specs=[
            pl.BlockSpec((BN, H), lambda i: (i, 0)),
            pl.BlockSpec((BN, H), lambda i: (i, 0)),
        ],
        out_shape=[
            jax.ShapeDtypeStruct((N, H), jnp.float32),
            jax.ShapeDtypeStruct((N, H), jnp.float32),
        ],
    )(feats, w1s, w1d)


@functools.lru_cache(maxsize=None)
def _make_gather():
    info = plsc.get_sparse_core_info()
    nc, ns = info.num_cores, info.num_subcores
    nw = nc * ns
    epw = E // nw
    nch = epw // CB

    def _gather_body(ta, tb, c4, src, dst, g_out, d2_out,
                     is0, id0, is1, id1, ra0, rb0, ra1, rb1, d20, d21,
                     c4_v, gs0, gs1):
        wid = lax.axis_index("s") * nc + lax.axis_index("c")
        base = wid * epw
        pltpu.sync_copy(c4, c4_v)

        def fire(k, isv, idv, ra, rb, gs):
            e0 = base + k * CB
            pltpu.sync_copy(src.at[pl.ds(e0, CB)], isv)
            pltpu.sync_copy(dst.at[pl.ds(e0, CB)], idv)
            pltpu.async_copy(ta.at[isv], ra, gs)
            pltpu.async_copy(tb.at[idv], rb, gs)

        def consume(k, isv, idv, ra, rb, gs, d2v):
            e0 = base + k * CB
            for gblk in range(CB // 16):
                sl = pl.ds(gblk * 16, 16)
                is4 = isv[sl] << 2
                id4 = idv[sl] << 2
                dx = (plsc.load_gather(c4_v, [is4])
                      - plsc.load_gather(c4_v, [id4]))
                dy = (plsc.load_gather(c4_v, [is4 + 1])
                      - plsc.load_gather(c4_v, [id4 + 1]))
                dz = (plsc.load_gather(c4_v, [is4 + 2])
                      - plsc.load_gather(c4_v, [id4 + 2]))
                d2v[sl] = dx * dx + dy * dy + dz * dz

            pltpu.make_async_copy(ta.at[pl.ds(0, CB)], ra, gs).wait()
            pltpu.make_async_copy(ta.at[pl.ds(0, CB)], rb, gs).wait()

            @plsc.parallel_loop(0, CB, unroll=2)
            def _row(i):
                for j in range(H // 16):
                    sl = pl.ds(j * 16, 16)
                    ra[i, sl] = ra[i, sl] + rb[i, sl]

            pltpu.sync_copy(ra, g_out.at[pl.ds(e0, CB)])
            pltpu.sync_copy(d2v, d2_out.at[pl.ds(e0, CB)])

        fire(0, is0, id0, ra0, rb0, gs0)

        def pair(p, carry):
            k0 = 2 * p
            fire(k0 + 1, is1, id1, ra1, rb1, gs1)
            consume(k0, is0, id0, ra0, rb0, gs0, d20)
            fire(k0 + 2, is0, id0, ra0, rb0, gs0)
            consume(k0 + 1, is1, id1, ra1, rb1, gs1, d21)
            return carry

        lax.fori_loop(0, (nch - 1) // 2, pair, 0)
        consume(nch - 1, is0, id0, ra0, rb0, gs0, d20)

    return functools.partial(
        pl.kernel,
        out_type=[
            jax.ShapeDtypeStruct((E, H), jnp.float32),
            jax.ShapeDtypeStruct((E,), jnp.float32),
        ],
        mesh=plsc.VectorSubcoreMesh(core_axis_name="c", subcore_axis_name="s"),
        compiler_params=pltpu.CompilerParams(needs_layout_passes=False),
        scratch_types=[
            pltpu.VMEM((CB,), jnp.int32),
            pltpu.VMEM((CB,), jnp.int32),
            pltpu.VMEM((CB,), jnp.int32),
            pltpu.VMEM((CB,), jnp.int32),
            pltpu.VMEM((CB, H), jnp.float32),
            pltpu.VMEM((CB, H), jnp.float32),
            pltpu.VMEM((CB, H), jnp.float32),
            pltpu.VMEM((CB, H), jnp.float32),
            pltpu.VMEM((CB,), jnp.float32),
            pltpu.VMEM((CB,), jnp.float32),
            pltpu.VMEM((4 * N,), jnp.float32),
            pltpu.SemaphoreType.DMA,
            pltpu.SemaphoreType.DMA,
        ],
    )(_gather_body)


def _mlp_body(g_ref, d2_ref, ef_ref, w1e_ref, w1m_ref, isg_ref, b1_ref,
              lng_ref, lnb_ref, w2_ref, b2_ref, o_ref):
    d2 = d2_ref[...]
    mag = jnp.exp(-d2 * isg_ref[...])
    u = (g_ref[...]
         + jnp.dot(ef_ref[...], w1e_ref[...], preferred_element_type=jnp.float32)
         + jnp.dot(mag, w1m_ref[...], preferred_element_type=jnp.float32)
         + b1_ref[...])
    h = jnp.where(u > 0, u, 0.01 * u)
    mu = jnp.mean(h, axis=-1, keepdims=True)
    hc = h - mu
    var = jnp.mean(hc * hc, axis=-1, keepdims=True)
    hn = hc * lax.rsqrt(var + 1e-5) * lng_ref[...] + lnb_ref[...]
    o_ref[...] = jnp.dot(hn, w2_ref[...], preferred_element_type=jnp.float32) + b2_ref[...]


def _mlp(g, d2, ef, w1e, w1m, isg, b1, lng, lnb, w2, b2):
    full = lambda i: (0, 0)
    return pl.pallas_call(
        _mlp_body,
        grid=(E // BE,),
        in_specs=[
            pl.BlockSpec((BE, H), lambda i: (i, 0)),
            pl.BlockSpec((BE, 1), lambda i: (i, 0)),
            pl.BlockSpec((BE, EF), lambda i: (i, 0)),
            pl.BlockSpec((EF, OUT), full),
            pl.BlockSpec((16, OUT), full),
            pl.BlockSpec((1, 16), full),
            pl.BlockSpec((1, OUT), full),
            pl.BlockSpec((1, OUT), full),
            pl.BlockSpec((1, OUT), full),
            pl.BlockSpec((OUT, OUT), full),
            pl.BlockSpec((1, OUT), full),
        ],
        out_specs=pl.BlockSpec((BE, OUT), lambda i: (i, 0)),
        out_shape=jax.ShapeDtypeStruct((E, OUT), jnp.float32),
    )(g, d2, ef, w1e, w1m, isg, b1, lng, lnb, w2, b2)


def kernel(coors_ligand, h_feats_ligand, original_ligand_node_features,
           original_edge_feats_ligand, orig_coors_ligand, coors_receptor,
           h_feats_receptor, original_receptor_node_features,
           original_edge_feats_receptor, orig_coors_receptor,
           edge_index_ligand, edge_index_receptor, W1, b1, ln_g, ln_b, W2, b2):
    w1s = W1[:H]
    w1d = W1[H:2 * H]
    w1e = W1[2 * H:2 * H + EF]
    w1m = jnp.concatenate([W1[2 * H + EF:], jnp.zeros((1, OUT), jnp.float32)], axis=0)
    isg = jnp.asarray(
        np.concatenate([1.0 / (1.5 ** np.arange(NSIG)), [0.0]]), jnp.float32
    ).reshape(1, 16)
    b1r = b1.reshape(1, OUT)
    b2r = b2.reshape(1, OUT)
    lngr = ln_g.reshape(1, OUT)
    lnbr = ln_b.reshape(1, OUT)

    c4_l = jnp.pad(coors_ligand, ((0, 0), (0, 1))).reshape(4 * N)
    c4_r = jnp.pad(coors_receptor, ((0, 0), (0, 1))).reshape(4 * N)

    ta_l, tb_l = _build_tables(h_feats_ligand, w1s, w1d)
    ta_r, tb_r = _build_tables(h_feats_receptor, w1s, w1d)

    gather = _make_gather()
    g_l, d2_l = gather(ta_l, tb_l, c4_l, edge_index_ligand[0], edge_index_ligand[1])
    g_r, d2_r = gather(ta_r, tb_r, c4_r, edge_index_receptor[0], edge_index_receptor[1])

    msg_ll = _mlp(g_l, d2_l.reshape(E, 1), original_edge_feats_ligand,
                  w1e, w1m, isg, b1r, lngr, lnbr, W2, b2r)
    msg_rr = _mlp(g_r, d2_r.reshape(E, 1), original_edge_feats_receptor,
                  w1e, w1m, isg, b1r, lngr, lnbr, W2, b2r)
    return (msg_ll, msg_rr)


# speedup vs baseline: 1.6095x; 1.1859x over previous
import functools

import jax
import jax.numpy as jnp
import numpy as np
from jax import lax
from jax.experimental import pallas as pl
from jax.experimental.pallas import tpu as pltpu
from jax.experimental.pallas import tpu_sc as plsc

N = 10000
E = 320000
H = 128
OUT = 128
EF = 16
NSIG = 15
DM = 144
CB = 80
BN = 2000
BE = 3200


def _tables_body(f_ref, ws_ref, wd_ref, ta_ref, tb_ref):
    f = f_ref[...]
    ta_ref[...] = jnp.dot(f, ws_ref[...], preferred_element_type=jnp.float32)
    tb_ref[...] = jnp.dot(f, wd_ref[...], preferred_element_type=jnp.float32)


def _build_tables(feats, w1s, w1d):
    return pl.pallas_call(
        _tables_body,
        grid=(N // BN,),
        in_specs=[
            pl.BlockSpec((BN, H), lambda i: (i, 0)),
            pl.BlockSpec((H, H), lambda i: (0, 0)),
            pl.BlockSpec((H, H), lambda i: (0, 0)),
        ],
        out_specs=[
            pl.BlockSpec((BN, H), lambda i: (i, 0)),
            pl.BlockSpec((BN, H), lambda i: (i, 0)),
        ],
        out_shape=[
            jax.ShapeDtypeStruct((N, H), jnp.float32),
            jax.ShapeDtypeStruct((N, H), jnp.float32),
        ],
    )(feats, w1s, w1d)


@functools.lru_cache(maxsize=None)
def _make_gather():
    info = plsc.get_sparse_core_info()
    nc, ns = info.num_cores, info.num_subcores
    nw = nc * ns
    epw = E // nw
    nch = epw // CB

    def _gather_body(ta, tb, c4, src, dst, nisg, m_out,
                     is0, id0, is1, id1, ra0, rb0, ra1, rb1, mb0, mb1,
                     c4_v, nisg_v, gs0, gs1):
        wid = lax.axis_index("s") * nc + lax.axis_index("c")
        base = wid * epw
        pltpu.sync_copy(c4, c4_v)
        pltpu.sync_copy(nisg, nisg_v)
        nisg_vec = nisg_v[...]

        def fire(k, isv, idv, ra, rb, gs):
            e0 = base + k * CB
            pltpu.sync_copy(src.at[pl.ds(e0, CB)], isv)
            pltpu.sync_copy(dst.at[pl.ds(e0, CB)], idv)
            pltpu.async_copy(ta.at[isv], ra, gs)
            pltpu.async_copy(tb.at[idv], rb, gs)

        def consume(k, isv, idv, ra, rb, mb, gs):
            e0 = base + k * CB
            for gblk in range(CB // 16):
                sl = pl.ds(gblk * 16, 16)
                is4 = isv[sl] << 2
                id4 = idv[sl] << 2
                dx = (plsc.load_gather(c4_v, [is4])
                      - plsc.load_gather(c4_v, [id4]))
                dy = (plsc.load_gather(c4_v, [is4 + 1])
                      - plsc.load_gather(c4_v, [id4 + 1]))
                dz = (plsc.load_gather(c4_v, [is4 + 2])
                      - plsc.load_gather(c4_v, [id4 + 2]))
                d2vec = dx * dx + dy * dy + dz * dz
                for e in range(16):
                    mb[gblk * 16 + e, pl.ds(H, 16)] = jnp.exp(
                        d2vec[e] * nisg_vec)

            pltpu.make_async_copy(ta.at[pl.ds(0, CB)], ra, gs).wait()
            pltpu.make_async_copy(ta.at[pl.ds(0, CB)], rb, gs).wait()

            @plsc.parallel_loop(0, CB, unroll=2)
            def _row(i):
                for j in range(H // 16):
                    sl = pl.ds(j * 16, 16)
                    mb[i, sl] = ra[i, sl] + rb[i, sl]

            pltpu.sync_copy(mb, m_out.at[pl.ds(e0, CB)])

        fire(0, is0, id0, ra0, rb0, gs0)

        def pair(p, carry):
            k0 = 2 * p
            fire(k0 + 1, is1, id1, ra1, rb1, gs1)
            consume(k0, is0, id0, ra0, rb0, mb0, gs0)
            fire(k0 + 2, is0, id0, ra0, rb0, gs0)
            consume(k0 + 1, is1, id1, ra1, rb1, mb1, gs1)
            return carry

        lax.fori_loop(0, (nch - 1) // 2, pair, 0)
        consume(nch - 1, is0, id0, ra0, rb0, mb0, gs0)

    return functools.partial(
        pl.kernel,
        out_type=jax.ShapeDtypeStruct((E, DM), jnp.float32),
        mesh=plsc.VectorSubcoreMesh(core_axis_name="c", subcore_axis_name="s"),
        compiler_params=pltpu.CompilerParams(needs_layout_passes=False),
        scratch_types=[
            pltpu.VMEM((CB,), jnp.int32),
            pltpu.VMEM((CB,), jnp.int32),
            pltpu.VMEM((CB,), jnp.int32),
            pltpu.VMEM((CB,), jnp.int32),
            pltpu.VMEM((CB, H), jnp.float32),
            pltpu.VMEM((CB, H), jnp.float32),
            pltpu.VMEM((CB, H), jnp.float32),
            pltpu.VMEM((CB, H), jnp.float32),
            pltpu.VMEM((CB, DM), jnp.float32),
            pltpu.VMEM((CB, DM), jnp.float32),
            pltpu.VMEM((4 * N,), jnp.float32),
            pltpu.VMEM((16,), jnp.float32),
            pltpu.SemaphoreType.DMA,
            pltpu.SemaphoreType.DMA,
        ],
    )(_gather_body)


def _mlp_body(m_ref, ef2_ref, w1ebd_ref, w1m_ref, b1_ref,
              lng_ref, lnb_ref, w2_ref, b2_ref, o_ref):
    m = m_ref[...]
    efc = jnp.dot(ef2_ref[...], w1ebd_ref[...],
                  preferred_element_type=jnp.float32)
    u = (m[:, :H]
         + efc.reshape(BE, OUT)
         + jnp.dot(m[:, H:], w1m_ref[...], preferred_element_type=jnp.float32)
         + b1_ref[...])
    h = jnp.where(u > 0, u, 0.01 * u)
    mu = jnp.mean(h, axis=-1, keepdims=True)
    hc = h - mu
    var = jnp.mean(hc * hc, axis=-1, keepdims=True)
    hn = hc * lax.rsqrt(var + 1e-5) * lng_ref[...] + lnb_ref[...]
    o_ref[...] = jnp.dot(hn, w2_ref[...], preferred_element_type=jnp.float32) + b2_ref[...]


def _mlp(m, ef2, w1ebd, w1m, b1, lng, lnb, w2, b2):
    full = lambda i: (0, 0)
    return pl.pallas_call(
        _mlp_body,
        grid=(E // BE,),
        in_specs=[
            pl.BlockSpec((BE, DM), lambda i: (i, 0)),
            pl.BlockSpec((BE // 8, H), lambda i: (i, 0)),
            pl.BlockSpec((H, 8 * OUT), full),
            pl.BlockSpec((16, OUT), full),
            pl.BlockSpec((1, OUT), full),
            pl.BlockSpec((1, OUT), full),
            pl.BlockSpec((1, OUT), full),
            pl.BlockSpec((OUT, OUT), full),
            pl.BlockSpec((1, OUT), full),
        ],
        out_specs=pl.BlockSpec((BE, OUT), lambda i: (i, 0)),
        out_shape=jax.ShapeDtypeStruct((E, OUT), jnp.float32),
    )(m, ef2, w1ebd, w1m, b1, lng, lnb, w2, b2)


def kernel(coors_ligand, h_feats_ligand, original_ligand_node_features,
           original_edge_feats_ligand, orig_coors_ligand, coors_receptor,
           h_feats_receptor, original_receptor_node_features,
           original_edge_feats_receptor, orig_coors_receptor,
           edge_index_ligand, edge_index_receptor, W1, b1, ln_g, ln_b, W2, b2):
    w1s = W1[:H]
    w1d = W1[H:2 * H]
    w1e = W1[2 * H:2 * H + EF]
    w1m = jnp.concatenate([W1[2 * H + EF:], jnp.zeros((1, OUT), jnp.float32)], axis=0)
    w1ebd = jnp.kron(jnp.eye(8, dtype=jnp.float32), w1e)
    nisg = jnp.asarray(
        np.concatenate([-1.0 / (1.5 ** np.arange(NSIG)), [0.0]]), jnp.float32)
    b1r = b1.reshape(1, OUT)
    b2r = b2.reshape(1, OUT)
    lngr = ln_g.reshape(1, OUT)
    lnbr = ln_b.reshape(1, OUT)

    c4_l = jnp.pad(coors_ligand, ((0, 0), (0, 1))).reshape(4 * N)
    c4_r = jnp.pad(coors_receptor, ((0, 0), (0, 1))).reshape(4 * N)
    ef2_l = original_edge_feats_ligand.reshape(E // 8, 128)
    ef2_r = original_edge_feats_receptor.reshape(E // 8, 128)

    ta_l, tb_l = _build_tables(h_feats_ligand, w1s, w1d)
    ta_r, tb_r = _build_tables(h_feats_receptor, w1s, w1d)

    gather = _make_gather()
    m_l = gather(ta_l, tb_l, c4_l, edge_index_ligand[0], edge_index_ligand[1], nisg)
    m_r = gather(ta_r, tb_r, c4_r, edge_index_receptor[0], edge_index_receptor[1], nisg)

    msg_ll = _mlp(m_l, ef2_l, w1ebd, w1m, b1r, lngr, lnbr, W2, b2r)
    msg_rr = _mlp(m_r, ef2_r, w1ebd, w1m, b1r, lngr, lnbr, W2, b2r)
    return (msg_ll, msg_rr)


# speedup vs baseline: 1.6173x; 1.0048x over previous
import functools

import jax
import jax.numpy as jnp
import numpy as np
from jax import lax
from jax.experimental import pallas as pl
from jax.experimental.pallas import tpu as pltpu
from jax.experimental.pallas import tpu_sc as plsc

N = 10000
E = 320000
H = 128
OUT = 128
EF = 16
NSIG = 15
DM = 144
CB = 80
BN = 2000
BE = 3200


def _tables_body(f_ref, ws_ref, wd_ref, ta_ref, tb_ref):
    f = f_ref[...]
    ta_ref[...] = jnp.dot(f, ws_ref[...], preferred_element_type=jnp.float32)
    tb_ref[...] = jnp.dot(f, wd_ref[...], preferred_element_type=jnp.float32)


def _build_tables(feats, w1s, w1d):
    return pl.pallas_call(
        _tables_body,
        grid=(N // BN,),
        in_specs=[
            pl.BlockSpec((BN, H), lambda i: (i, 0)),
            pl.BlockSpec((H, H), lambda i: (0, 0)),
            pl.BlockSpec((H, H), lambda i: (0, 0)),
        ],
        out_specs=[
            pl.BlockSpec((BN, H), lambda i: (i, 0)),
            pl.BlockSpec((BN, H), lambda i: (i, 0)),
        ],
        out_shape=[
            jax.ShapeDtypeStruct((N, H), jnp.float32),
            jax.ShapeDtypeStruct((N, H), jnp.float32),
        ],
    )(feats, w1s, w1d)


@functools.lru_cache(maxsize=None)
def _make_gather():
    info = plsc.get_sparse_core_info()
    nc, ns = info.num_cores, info.num_subcores
    nw = nc * ns
    epw = E // nw
    nch = epw // CB

    def _gather_body(ta, tb, c4, src, dst, m_out,
                     is0, id0, is1, id1, ra0, rb0, ra1, rb1, mg0, mg1,
                     c4_v, gs0, gs1):
        wid = lax.axis_index("s") * nc + lax.axis_index("c")
        base = wid * epw
        pltpu.sync_copy(c4, c4_v)

        def fire(k, isv, idv, ra, rb, gs):
            e0 = base + k * CB
            pltpu.sync_copy(src.at[pl.ds(e0, CB)], isv)
            pltpu.sync_copy(dst.at[pl.ds(e0, CB)], idv)
            pltpu.async_copy(ta.at[isv], ra, gs)
            pltpu.async_copy(tb.at[idv], rb, gs)

        def consume(k, isv, idv, ra, rb, mg, gs):
            e0 = base + k * CB
            for gblk in range(CB // 16):
                sl = pl.ds(gblk * 16, 16)
                is4 = isv[sl] << 2
                id4 = idv[sl] << 2
                dx = (plsc.load_gather(c4_v, [is4])
                      - plsc.load_gather(c4_v, [id4]))
                dy = (plsc.load_gather(c4_v, [is4 + 1])
                      - plsc.load_gather(c4_v, [id4 + 1]))
                dz = (plsc.load_gather(c4_v, [is4 + 2])
                      - plsc.load_gather(c4_v, [id4 + 2]))
                d2vec = dx * dx + dy * dy + dz * dz
                for e in range(16):
                    mg[gblk * 16 + e, :] = jnp.full((16,), d2vec[e],
                                                    jnp.float32)

            pltpu.make_async_copy(ta.at[pl.ds(0, CB)], ra, gs).wait()
            pltpu.make_async_copy(ta.at[pl.ds(0, CB)], rb, gs).wait()

            @plsc.parallel_loop(0, CB, unroll=2)
            def _row(i):
                for j in range(H // 16):
                    sl = pl.ds(j * 16, 16)
                    ra[i, sl] = ra[i, sl] + rb[i, sl]

            pltpu.sync_copy(ra, m_out.at[pl.ds(e0, CB), pl.ds(0, H)])
            pltpu.sync_copy(mg, m_out.at[pl.ds(e0, CB), pl.ds(H, 16)])

        fire(0, is0, id0, ra0, rb0, gs0)

        def pair(p, carry):
            k0 = 2 * p
            fire(k0 + 1, is1, id1, ra1, rb1, gs1)
            consume(k0, is0, id0, ra0, rb0, mg0, gs0)
            fire(k0 + 2, is0, id0, ra0, rb0, gs0)
            consume(k0 + 1, is1, id1, ra1, rb1, mg1, gs1)
            return carry

        lax.fori_loop(0, (nch - 1) // 2, pair, 0)
        consume(nch - 1, is0, id0, ra0, rb0, mg0, gs0)

    return functools.partial(
        pl.kernel,
        out_type=jax.ShapeDtypeStruct((E, DM), jnp.float32),
        mesh=plsc.VectorSubcoreMesh(core_axis_name="c", subcore_axis_name="s"),
        compiler_params=pltpu.CompilerParams(needs_layout_passes=False),
        scratch_types=[
            pltpu.VMEM((CB,), jnp.int32),
            pltpu.VMEM((CB,), jnp.int32),
            pltpu.VMEM((CB,), jnp.int32),
            pltpu.VMEM((CB,), jnp.int32),
            pltpu.VMEM((CB, H), jnp.float32),
            pltpu.VMEM((CB, H), jnp.float32),
            pltpu.VMEM((CB, H), jnp.float32),
            pltpu.VMEM((CB, H), jnp.float32),
            pltpu.VMEM((CB, 16), jnp.float32),
            pltpu.VMEM((CB, 16), jnp.float32),
            pltpu.VMEM((4 * N,), jnp.float32),
            pltpu.SemaphoreType.DMA,
            pltpu.SemaphoreType.DMA,
        ],
    )(_gather_body)


def _mlp_body(m_ref, ef2_ref, w1ebd_ref, w1m_ref, nisg_ref, b1_ref,
              lng_ref, lnb_ref, w2_ref, b2_ref, o_ref):
    m = m_ref[...]
    efc = jnp.dot(ef2_ref[...], w1ebd_ref[...],
                  preferred_element_type=jnp.float32)
    mag = jnp.exp(m[:, H:] * nisg_ref[...])
    u = (m[:, :H]
         + efc.reshape(BE, OUT)
         + jnp.dot(mag, w1m_ref[...], preferred_element_type=jnp.float32)
         + b1_ref[...])
    h = jnp.where(u > 0, u, 0.01 * u)
    mu = jnp.mean(h, axis=-1, keepdims=True)
    hc = h - mu
    var = jnp.mean(hc * hc, axis=-1, keepdims=True)
    hn = hc * lax.rsqrt(var + 1e-5) * lng_ref[...] + lnb_ref[...]
    o_ref[...] = jnp.dot(hn, w2_ref[...], preferred_element_type=jnp.float32) + b2_ref[...]


def _mlp(m, ef2, w1ebd, w1m, nisg, b1, lng, lnb, w2, b2):
    full = lambda i: (0, 0)
    return pl.pallas_call(
        _mlp_body,
        grid=(E // BE,),
        in_specs=[
            pl.BlockSpec((BE, DM), lambda i: (i, 0)),
            pl.BlockSpec((BE // 8, H), lambda i: (i, 0)),
            pl.BlockSpec((H, 8 * OUT), full),
            pl.BlockSpec((16, OUT), full),
            pl.BlockSpec((1, 16), full),
            pl.BlockSpec((1, OUT), full),
            pl.BlockSpec((1, OUT), full),
            pl.BlockSpec((1, OUT), full),
            pl.BlockSpec((OUT, OUT), full),
            pl.BlockSpec((1, OUT), full),
        ],
        out_specs=pl.BlockSpec((BE, OUT), lambda i: (i, 0)),
        out_shape=jax.ShapeDtypeStruct((E, OUT), jnp.float32),
    )(m, ef2, w1ebd, w1m, nisg, b1, lng, lnb, w2, b2)


def kernel(coors_ligand, h_feats_ligand, original_ligand_node_features,
           original_edge_feats_ligand, orig_coors_ligand, coors_receptor,
           h_feats_receptor, original_receptor_node_features,
           original_edge_feats_receptor, orig_coors_receptor,
           edge_index_ligand, edge_index_receptor, W1, b1, ln_g, ln_b, W2, b2):
    w1s = W1[:H]
    w1d = W1[H:2 * H]
    w1e = W1[2 * H:2 * H + EF]
    w1m = jnp.concatenate([W1[2 * H + EF:], jnp.zeros((1, OUT), jnp.float32)], axis=0)
    w1ebd = jnp.kron(jnp.eye(8, dtype=jnp.float32), w1e)
    nisg = jnp.asarray(
        np.concatenate([-1.0 / (1.5 ** np.arange(NSIG)), [0.0]]), jnp.float32
    ).reshape(1, 16)
    b1r = b1.reshape(1, OUT)
    b2r = b2.reshape(1, OUT)
    lngr = ln_g.reshape(1, OUT)
    lnbr = ln_b.reshape(1, OUT)

    c4_l = jnp.pad(coors_ligand, ((0, 0), (0, 1))).reshape(4 * N)
    c4_r = jnp.pad(coors_receptor, ((0, 0), (0, 1))).reshape(4 * N)
    ef2_l = original_edge_feats_ligand.reshape(E // 8, 128)
    ef2_r = original_edge_feats_receptor.reshape(E // 8, 128)

    ta_l, tb_l = _build_tables(h_feats_ligand, w1s, w1d)
    ta_r, tb_r = _build_tables(h_feats_receptor, w1s, w1d)

    gather = _make_gather()
    m_l = gather(ta_l, tb_l, c4_l, edge_index_ligand[0], edge_index_ligand[1])
    m_r = gather(ta_r, tb_r, c4_r, edge_index_receptor[0], edge_index_receptor[1])

    msg_ll = _mlp(m_l, ef2_l, w1ebd, w1m, nisg, b1r, lngr, lnbr, W2, b2r)
    msg_rr = _mlp(m_r, ef2_r, w1ebd, w1m, nisg, b1r, lngr, lnbr, W2, b2r)
    return (msg_ll, msg_rr)


# speedup vs baseline: 1.7489x; 1.0814x over previous
import functools

import jax
import jax.numpy as jnp
import numpy as np
from jax import lax
from jax.experimental import pallas as pl
from jax.experimental.pallas import tpu as pltpu
from jax.experimental.pallas import tpu_sc as plsc

N = 10000
E = 320000
H = 128
OUT = 128
EF = 16
NSIG = 15
DM = 144
CB = 80
BN = 2000
BE = 3200


def _tables_body(f_ref, ws_ref, wd_ref, ta_ref, tb_ref):
    f = f_ref[...]
    ta_ref[...] = jnp.dot(f, ws_ref[...], preferred_element_type=jnp.float32)
    tb_ref[...] = jnp.dot(f, wd_ref[...], preferred_element_type=jnp.float32)


def _build_tables(feats, w1s, w1d):
    return pl.pallas_call(
        _tables_body,
        grid=(N // BN,),
        in_specs=[
            pl.BlockSpec((BN, H), lambda i: (i, 0)),
            pl.BlockSpec((H, H), lambda i: (0, 0)),
            pl.BlockSpec((H, H), lambda i: (0, 0)),
        ],
        out_specs=[
            pl.BlockSpec((BN, H), lambda i: (i, 0)),
            pl.BlockSpec((BN, H), lambda i: (i, 0)),
        ],
        out_shape=[
            jax.ShapeDtypeStruct((N, H), jnp.float32),
            jax.ShapeDtypeStruct((N, H), jnp.float32),
        ],
    )(feats, w1s, w1d)


@functools.lru_cache(maxsize=None)
def _make_gather():
    info = plsc.get_sparse_core_info()
    nc, ns = info.num_cores, info.num_subcores
    nw = nc * ns
    epw = E // nw
    nch = epw // CB

    def _gather_body(ta, tb, c4, src, dst, m_out,
                     ix0, ix1, ra0, rb0, ra1, rb1, mb0, mb1,
                     c4_v, gs0, gs1, ss0, ss1):
        wid = lax.axis_index("s") * nc + lax.axis_index("c")
        base = wid * epw
        pltpu.sync_copy(c4, c4_v)

        def fire(k, ix, ra, rb, gs):
            e0 = base + k * CB
            pltpu.sync_copy(src.at[pl.ds(e0, CB)], ix.at[0])
            pltpu.sync_copy(dst.at[pl.ds(e0, CB)], ix.at[1])
            pltpu.async_copy(ta.at[ix.at[0]], ra, gs)
            pltpu.async_copy(tb.at[ix.at[1]], rb, gs)

        def consume(k, ix, ra, rb, mb, gs, ss, wait_cond):
            e0 = base + k * CB
            if wait_cond is True:
                pltpu.make_async_copy(m_out.at[pl.ds(0, CB)], mb, ss).wait()
            else:
                @pl.when(wait_cond)
                def _():
                    pltpu.make_async_copy(m_out.at[pl.ds(0, CB)], mb, ss).wait()

            for gblk in range(CB // 16):
                sl = pl.ds(gblk * 16, 16)
                is4 = ix[0, sl] << 2
                id4 = ix[1, sl] << 2
                dx = (plsc.load_gather(c4_v, [is4])
                      - plsc.load_gather(c4_v, [id4]))
                dy = (plsc.load_gather(c4_v, [is4 + 1])
                      - plsc.load_gather(c4_v, [id4 + 1]))
                dz = (plsc.load_gather(c4_v, [is4 + 2])
                      - plsc.load_gather(c4_v, [id4 + 2]))
                d2vec = dx * dx + dy * dy + dz * dz
                for e in range(16):
                    mb[gblk * 16 + e, pl.ds(H, 16)] = jnp.full(
                        (16,), d2vec[e], jnp.float32)

            pltpu.make_async_copy(ta.at[pl.ds(0, CB)], ra, gs).wait()
            pltpu.make_async_copy(ta.at[pl.ds(0, CB)], rb, gs).wait()

            @plsc.parallel_loop(0, CB, unroll=2)
            def _row(i):
                for j in range(H // 16):
                    sl = pl.ds(j * 16, 16)
                    mb[i, sl] = ra[i, sl] + rb[i, sl]

            pltpu.async_copy(mb, m_out.at[pl.ds(e0, CB)], ss)

        fire(0, ix0, ra0, rb0, gs0)

        def pair(p, carry):
            k0 = 2 * p
            fire(k0 + 1, ix1, ra1, rb1, gs1)
            consume(k0, ix0, ra0, rb0, mb0, gs0, ss0, p >= 1)
            fire(k0 + 2, ix0, ra0, rb0, gs0)
            consume(k0 + 1, ix1, ra1, rb1, mb1, gs1, ss1, p >= 1)
            return carry

        lax.fori_loop(0, (nch - 1) // 2, pair, 0)
        consume(nch - 1, ix0, ra0, rb0, mb0, gs0, ss0, True)

        pltpu.make_async_copy(m_out.at[pl.ds(0, CB)], mb1, ss1).wait()
        pltpu.make_async_copy(m_out.at[pl.ds(0, CB)], mb0, ss0).wait()

    return functools.partial(
        pl.kernel,
        out_type=jax.ShapeDtypeStruct((E, DM), jnp.float32),
        mesh=plsc.VectorSubcoreMesh(core_axis_name="c", subcore_axis_name="s"),
        compiler_params=pltpu.CompilerParams(needs_layout_passes=False),
        scratch_types=[
            pltpu.VMEM((2, CB), jnp.int32),
            pltpu.VMEM((2, CB), jnp.int32),
            pltpu.VMEM((CB, H), jnp.float32),
            pltpu.VMEM((CB, H), jnp.float32),
            pltpu.VMEM((CB, H), jnp.float32),
            pltpu.VMEM((CB, H), jnp.float32),
            pltpu.VMEM((CB, DM), jnp.float32),
            pltpu.VMEM((CB, DM), jnp.float32),
            pltpu.VMEM((4 * N,), jnp.float32),
            pltpu.SemaphoreType.DMA,
            pltpu.SemaphoreType.DMA,
            pltpu.SemaphoreType.DMA,
            pltpu.SemaphoreType.DMA,
        ],
    )(_gather_body)


def _mlp_body(m_ref, ef2_ref, w1ebd_ref, w1m_ref, nisg_ref, b1_ref,
              lng_ref, lnb_ref, w2_ref, b2_ref, o_ref):
    m = m_ref[...]
    efc = jnp.dot(ef2_ref[...], w1ebd_ref[...],
                  preferred_element_type=jnp.float32)
    mag = jnp.exp(m[:, H:] * nisg_ref[...])
    u = (m[:, :H]
         + efc.reshape(BE, OUT)
         + jnp.dot(mag, w1m_ref[...], preferred_element_type=jnp.float32)
         + b1_ref[...])
    h = jnp.where(u > 0, u, 0.01 * u)
    mu = jnp.mean(h, axis=-1, keepdims=True)
    hc = h - mu
    var = jnp.mean(hc * hc, axis=-1, keepdims=True)
    hn = hc * lax.rsqrt(var + 1e-5) * lng_ref[...] + lnb_ref[...]
    o_ref[...] = jnp.dot(hn, w2_ref[...], preferred_element_type=jnp.float32) + b2_ref[...]


def _mlp(m, ef2, w1ebd, w1m, nisg, b1, lng, lnb, w2, b2):
    full = lambda i: (0, 0)
    return pl.pallas_call(
        _mlp_body,
        grid=(E // BE,),
        in_specs=[
            pl.BlockSpec((BE, DM), lambda i: (i, 0)),
            pl.BlockSpec((BE // 8, H), lambda i: (i, 0)),
            pl.BlockSpec((H, 8 * OUT), full),
            pl.BlockSpec((16, OUT), full),
            pl.BlockSpec((1, 16), full),
            pl.BlockSpec((1, OUT), full),
            pl.BlockSpec((1, OUT), full),
            pl.BlockSpec((1, OUT), full),
            pl.BlockSpec((OUT, OUT), full),
            pl.BlockSpec((1, OUT), full),
        ],
        out_specs=pl.BlockSpec((BE, OUT), lambda i: (i, 0)),
        out_shape=jax.ShapeDtypeStruct((E, OUT), jnp.float32),
    )(m, ef2, w1ebd, w1m, nisg, b1, lng, lnb, w2, b2)


def kernel(coors_ligand, h_feats_ligand, original_ligand_node_features,
           original_edge_feats_ligand, orig_coors_ligand, coors_receptor,
           h_feats_receptor, original_receptor_node_features,
           original_edge_feats_receptor, orig_coors_receptor,
           edge_index_ligand, edge_index_receptor, W1, b1, ln_g, ln_b, W2, b2):
    w1s = W1[:H]
    w1d = W1[H:2 * H]
    w1e = W1[2 * H:2 * H + EF]
    w1m = jnp.concatenate([W1[2 * H + EF:], jnp.zeros((1, OUT), jnp.float32)], axis=0)
    w1ebd = jnp.kron(jnp.eye(8, dtype=jnp.float32), w1e)
    nisg = jnp.asarray(
        np.concatenate([-1.0 / (1.5 ** np.arange(NSIG)), [0.0]]), jnp.float32
    ).reshape(1, 16)
    b1r = b1.reshape(1, OUT)
    b2r = b2.reshape(1, OUT)
    lngr = ln_g.reshape(1, OUT)
    lnbr = ln_b.reshape(1, OUT)

    c4_l = jnp.pad(coors_ligand, ((0, 0), (0, 1))).reshape(4 * N)
    c4_r = jnp.pad(coors_receptor, ((0, 0), (0, 1))).reshape(4 * N)
    ef2_l = original_edge_feats_ligand.reshape(E // 8, 128)
    ef2_r = original_edge_feats_receptor.reshape(E // 8, 128)

    ta_l, tb_l = _build_tables(h_feats_ligand, w1s, w1d)
    ta_r, tb_r = _build_tables(h_feats_receptor, w1s, w1d)

    gather = _make_gather()
    m_l = gather(ta_l, tb_l, c4_l, edge_index_ligand[0], edge_index_ligand[1])
    m_r = gather(ta_r, tb_r, c4_r, edge_index_receptor[0], edge_index_receptor[1])

    msg_ll = _mlp(m_l, ef2_l, w1ebd, w1m, nisg, b1r, lngr, lnbr, W2, b2r)
    msg_rr = _mlp(m_r, ef2_r, w1ebd, w1m, nisg, b1r, lngr, lnbr, W2, b2r)
    return (msg_ll, msg_rr)


# speedup vs baseline: 1.7514x; 1.0014x over previous
import functools

import jax
import jax.numpy as jnp
import numpy as np
from jax import lax
from jax.experimental import pallas as pl
from jax.experimental.pallas import tpu as pltpu
from jax.experimental.pallas import tpu_sc as plsc

N = 10000
E = 320000
H = 128
OUT = 128
EF = 16
NSIG = 15
DM = 144
CB = 80
BN = 2000
BE = 3200


def _tables_body(f_ref, ws_ref, wd_ref, ta_ref, tb_ref):
    f = f_ref[...]
    ta_ref[...] = jnp.dot(f, ws_ref[...], preferred_element_type=jnp.float32)
    tb_ref[...] = jnp.dot(f, wd_ref[...], preferred_element_type=jnp.float32)


def _build_tables(feats, w1s, w1d):
    return pl.pallas_call(
        _tables_body,
        grid=(N // BN,),
        in_specs=[
            pl.BlockSpec((BN, H), lambda i: (i, 0)),
            pl.BlockSpec((H, H), lambda i: (0, 0)),
            pl.BlockSpec((H, H), lambda i: (0, 0)),
        ],
        out_specs=[
            pl.BlockSpec((BN, H), lambda i: (i, 0)),
            pl.BlockSpec((BN, H), lambda i: (i, 0)),
        ],
        out_shape=[
            jax.ShapeDtypeStruct((N, H), jnp.float32),
            jax.ShapeDtypeStruct((N, H), jnp.float32),
        ],
    )(feats, w1s, w1d)


@functools.lru_cache(maxsize=None)
def _make_gather():
    info = plsc.get_sparse_core_info()
    nc, ns = info.num_cores, info.num_subcores
    nw = nc * ns
    epw = E // nw
    nch = epw // CB

    def _gather_body(ta, tb, c4, src, dst, m_out,
                     ix0, ix1, ra0, rb0, ra1, rb1, mb0, mb1,
                     c4_v, gs0, gs1, ss0, ss1):
        wid = lax.axis_index("s") * nc + lax.axis_index("c")
        base = wid * epw
        pltpu.sync_copy(c4, c4_v)

        def fire(k, ix, ra, rb, gs):
            e0 = base + k * CB
            pltpu.sync_copy(src.at[pl.ds(e0, CB)], ix.at[0])
            pltpu.sync_copy(dst.at[pl.ds(e0, CB)], ix.at[1])
            pltpu.async_copy(ta.at[ix.at[0]], ra, gs)
            pltpu.async_copy(tb.at[ix.at[1]], rb, gs)

        def consume(k, ix, ra, rb, mb, gs, ss, wait_cond):
            e0 = base + k * CB
            if wait_cond is True:
                pltpu.make_async_copy(m_out.at[pl.ds(0, CB)], mb, ss).wait()
            else:
                @pl.when(wait_cond)
                def _():
                    pltpu.make_async_copy(m_out.at[pl.ds(0, CB)], mb, ss).wait()

            for gblk in range(CB // 16):
                sl = pl.ds(gblk * 16, 16)
                is4 = ix[0, sl] << 2
                id4 = ix[1, sl] << 2
                dx = (plsc.load_gather(c4_v, [is4])
                      - plsc.load_gather(c4_v, [id4]))
                dy = (plsc.load_gather(c4_v, [is4 + 1])
                      - plsc.load_gather(c4_v, [id4 + 1]))
                dz = (plsc.load_gather(c4_v, [is4 + 2])
                      - plsc.load_gather(c4_v, [id4 + 2]))
                d2vec = dx * dx + dy * dy + dz * dz
                for e in range(16):
                    mb[gblk * 16 + e, pl.ds(H, 16)] = jnp.full(
                        (16,), d2vec[e], jnp.float32)

            pltpu.make_async_copy(ta.at[pl.ds(0, CB)], ra, gs).wait()
            pltpu.make_async_copy(ta.at[pl.ds(0, CB)], rb, gs).wait()

            @plsc.parallel_loop(0, CB, unroll=4)
            def _row(i):
                for j in range(H // 16):
                    sl = pl.ds(j * 16, 16)
                    mb[i, sl] = ra[i, sl] + rb[i, sl]

            pltpu.async_copy(mb, m_out.at[pl.ds(e0, CB)], ss)

        fire(0, ix0, ra0, rb0, gs0)

        def pair(p, carry):
            k0 = 2 * p
            fire(k0 + 1, ix1, ra1, rb1, gs1)
            consume(k0, ix0, ra0, rb0, mb0, gs0, ss0, p >= 1)
            fire(k0 + 2, ix0, ra0, rb0, gs0)
            consume(k0 + 1, ix1, ra1, rb1, mb1, gs1, ss1, p >= 1)
            return carry

        lax.fori_loop(0, (nch - 1) // 2, pair, 0)
        consume(nch - 1, ix0, ra0, rb0, mb0, gs0, ss0, True)

        pltpu.make_async_copy(m_out.at[pl.ds(0, CB)], mb1, ss1).wait()
        pltpu.make_async_copy(m_out.at[pl.ds(0, CB)], mb0, ss0).wait()

    return functools.partial(
        pl.kernel,
        out_type=jax.ShapeDtypeStruct((E, DM), jnp.float32),
        mesh=plsc.VectorSubcoreMesh(core_axis_name="c", subcore_axis_name="s"),
        compiler_params=pltpu.CompilerParams(needs_layout_passes=False),
        scratch_types=[
            pltpu.VMEM((2, CB), jnp.int32),
            pltpu.VMEM((2, CB), jnp.int32),
            pltpu.VMEM((CB, H), jnp.float32),
            pltpu.VMEM((CB, H), jnp.float32),
            pltpu.VMEM((CB, H), jnp.float32),
            pltpu.VMEM((CB, H), jnp.float32),
            pltpu.VMEM((CB, DM), jnp.float32),
            pltpu.VMEM((CB, DM), jnp.float32),
            pltpu.VMEM((4 * N,), jnp.float32),
            pltpu.SemaphoreType.DMA,
            pltpu.SemaphoreType.DMA,
            pltpu.SemaphoreType.DMA,
            pltpu.SemaphoreType.DMA,
        ],
    )(_gather_body)


def _mlp_body(m_ref, ef2_ref, w1ebd_ref, w1m_ref, nisg_ref, b1_ref,
              lng_ref, lnb_ref, w2_ref, b2_ref, o_ref):
    m = m_ref[...]
    efc = jnp.dot(ef2_ref[...], w1ebd_ref[...],
                  preferred_element_type=jnp.float32)
    mag = jnp.exp(m[:, H:] * nisg_ref[...])
    u = (m[:, :H]
         + efc.reshape(BE, OUT)
         + jnp.dot(mag, w1m_ref[...], preferred_element_type=jnp.float32)
         + b1_ref[...])
    h = jnp.where(u > 0, u, 0.01 * u)
    mu = jnp.mean(h, axis=-1, keepdims=True)
    hc = h - mu
    var = jnp.mean(hc * hc, axis=-1, keepdims=True)
    hn = hc * lax.rsqrt(var + 1e-5) * lng_ref[...] + lnb_ref[...]
    o_ref[...] = jnp.dot(hn, w2_ref[...], preferred_element_type=jnp.float32) + b2_ref[...]


def _mlp(m, ef2, w1ebd, w1m, nisg, b1, lng, lnb, w2, b2):
    full = lambda i: (0, 0)
    return pl.pallas_call(
        _mlp_body,
        grid=(E // BE,),
        in_specs=[
            pl.BlockSpec((BE, DM), lambda i: (i, 0)),
            pl.BlockSpec((BE // 8, H), lambda i: (i, 0)),
            pl.BlockSpec((H, 8 * OUT), full),
            pl.BlockSpec((16, OUT), full),
            pl.BlockSpec((1, 16), full),
            pl.BlockSpec((1, OUT), full),
            pl.BlockSpec((1, OUT), full),
            pl.BlockSpec((1, OUT), full),
            pl.BlockSpec((OUT, OUT), full),
            pl.BlockSpec((1, OUT), full),
        ],
        out_specs=pl.BlockSpec((BE, OUT), lambda i: (i, 0)),
        out_shape=jax.ShapeDtypeStruct((E, OUT), jnp.float32),
    )(m, ef2, w1ebd, w1m, nisg, b1, lng, lnb, w2, b2)


def kernel(coors_ligand, h_feats_ligand, original_ligand_node_features,
           original_edge_feats_ligand, orig_coors_ligand, coors_receptor,
           h_feats_receptor, original_receptor_node_features,
           original_edge_feats_receptor, orig_coors_receptor,
           edge_index_ligand, edge_index_receptor, W1, b1, ln_g, ln_b, W2, b2):
    w1s = W1[:H]
    w1d = W1[H:2 * H]
    w1e = W1[2 * H:2 * H + EF]
    w1m = jnp.concatenate([W1[2 * H + EF:], jnp.zeros((1, OUT), jnp.float32)], axis=0)
    w1ebd = jnp.kron(jnp.eye(8, dtype=jnp.float32), w1e)
    nisg = jnp.asarray(
        np.concatenate([-1.0 / (1.5 ** np.arange(NSIG)), [0.0]]), jnp.float32
    ).reshape(1, 16)
    b1r = b1.reshape(1, OUT)
    b2r = b2.reshape(1, OUT)
    lngr = ln_g.reshape(1, OUT)
    lnbr = ln_b.reshape(1, OUT)

    c4_l = jnp.pad(coors_ligand, ((0, 0), (0, 1))).reshape(4 * N)
    c4_r = jnp.pad(coors_receptor, ((0, 0), (0, 1))).reshape(4 * N)
    ef2_l = original_edge_feats_ligand.reshape(E // 8, 128)
    ef2_r = original_edge_feats_receptor.reshape(E // 8, 128)

    ta_l, tb_l = _build_tables(h_feats_ligand, w1s, w1d)
    ta_r, tb_r = _build_tables(h_feats_receptor, w1s, w1d)

    gather = _make_gather()
    m_l = gather(ta_l, tb_l, c4_l, edge_index_ligand[0], edge_index_ligand[1])
    m_r = gather(ta_r, tb_r, c4_r, edge_index_receptor[0], edge_index_receptor[1])

    msg_ll = _mlp(m_l, ef2_l, w1ebd, w1m, nisg, b1r, lngr, lnbr, W2, b2r)
    msg_rr = _mlp(m_r, ef2_r, w1ebd, w1m, nisg, b1r, lngr, lnbr, W2, b2r)
    return (msg_ll, msg_rr)
